# baseline (speedup 1.0000x reference)
import functools

import jax
import jax.numpy as jnp
from jax import lax
from jax.experimental import pallas as pl
from jax.experimental.pallas import tpu as pltpu
from jax.experimental.pallas import tpu_sc as plsc

BATCH = 16384
IN_DIM = 128
EMB_DIM = 18
RMIN = 64
RMAX = 71
NPATCH = RMAX - RMIN

L = 16
NC = 2
NS = 16
NW = NC * NS
ROWS_PER_W = BATCH // NW
CHUNK = 128
NCHUNKS = ROWS_PER_W // CHUNK


def _sc_planes(x_hbm, cb_hbm, out_hbm, x_v, cb_v, patch_v, sem, psem, xsem):
    wid = lax.axis_index("s") * NC + lax.axis_index("c")
    pltpu.sync_copy(cb_hbm, cb_v)

    iota = lax.iota(jnp.int32, L)
    mask_patch = iota < NPATCH

    row0 = wid * ROWS_PER_W
    pltpu.async_copy(x_hbm.at[pl.ds(row0, CHUNK)], x_v.at[0], xsem)

    def chunk_body(c, carry):
        base = row0 + c * CHUNK
        p = lax.rem(c, 2)
        pltpu.make_async_copy(
            x_hbm.at[pl.ds(base, CHUNK)], x_v.at[p], xsem
        ).wait()

        @pl.when(c + 1 < NCHUNKS)
        def _():
            pltpu.async_copy(
                x_hbm.at[pl.ds(base + CHUNK, CHUNK)],
                x_v.at[lax.rem(c + 1, 2)],
                xsem,
            )

        copies = []
        for j in range(EMB_DIM):
            copies.append(
                pltpu.async_copy(
                    x_v.at[p],
                    out_hbm.at[j, pl.ds(base, CHUNK)],
                    sem,
                )
            )

        @pl.when(c > 0)
        def _():
            pltpu.make_async_copy(
                patch_v,
                out_hbm.at[
                    pl.ds(0, EMB_DIM), pl.ds(base, CHUNK), pl.ds(RMIN, L)
                ],
                psem,
            ).wait()

        def row_body(r, rcarry):
            xv = x_v[p, r, pl.ds(RMIN, L)]
            ci = xv.astype(jnp.int32) * EMB_DIM
            for j in range(EMB_DIM):
                emb = plsc.load_gather(cb_v, [ci + j])
                patch_v[j, r] = jnp.where(mask_patch, emb, xv)
            return rcarry

        lax.fori_loop(0, CHUNK, row_body, 0)

        for cp in copies:
            cp.wait()
        pltpu.async_copy(
            patch_v,
            out_hbm.at[pl.ds(0, EMB_DIM), pl.ds(base, CHUNK), pl.ds(RMIN, L)],
            psem,
        )
        return carry

    lax.fori_loop(0, NCHUNKS, chunk_body, 0)
    last = row0 + (NCHUNKS - 1) * CHUNK
    pltpu.make_async_copy(
        patch_v,
        out_hbm.at[pl.ds(0, EMB_DIM), pl.ds(last, CHUNK), pl.ds(RMIN, L)],
        psem,
    ).wait()


@jax.jit
def _run(x, cbf):
    fn = functools.partial(
        pl.kernel,
        mesh=plsc.VectorSubcoreMesh(core_axis_name="c", subcore_axis_name="s"),
        compiler_params=pltpu.CompilerParams(
            needs_layout_passes=False, use_tc_tiling_on_sc=False
        ),
        out_type=jax.ShapeDtypeStruct((EMB_DIM, BATCH, IN_DIM), jnp.float32),
        scratch_types=[
            pltpu.VMEM((2, CHUNK, IN_DIM), jnp.float32),
            pltpu.VMEM((52 * EMB_DIM,), jnp.float32),
            pltpu.VMEM((EMB_DIM, CHUNK, L), jnp.float32),
            pltpu.SemaphoreType.DMA,
            pltpu.SemaphoreType.DMA,
            pltpu.SemaphoreType.DMA,
        ],
    )(_sc_planes)
    return fn(x, cbf)


def kernel(x, card_buffer):
    if x.ndim == 3:
        x = x[:, 0, :]
    cbf = card_buffer.reshape(52 * EMB_DIM)
    out = _run(x, cbf)
    return out.transpose(1, 2, 0)


# speedup vs baseline: 1.0530x; 1.0530x over previous
import functools

import jax
import jax.numpy as jnp
from jax import lax
from jax.experimental import pallas as pl
from jax.experimental.pallas import tpu as pltpu
from jax.experimental.pallas import tpu_sc as plsc

BATCH = 16384
IN_DIM = 128
EMB_DIM = 18
RMIN = 64
RMAX = 71
NPATCH = RMAX - RMIN

L = 16
NC = 2
NS = 16
NW = NC * NS
ROWS_PER_W = BATCH // NW
CHUNK = 256
NCHUNKS = ROWS_PER_W // CHUNK


def _sc_planes(x_hbm, cb_hbm, out_hbm, x_v, cb_v, patch_v, sem, psem):
    wid = lax.axis_index("s") * NC + lax.axis_index("c")
    pltpu.sync_copy(cb_hbm, cb_v)

    iota = lax.iota(jnp.int32, L)
    mask_patch = iota < NPATCH

    row0 = wid * ROWS_PER_W

    def chunk_body(c, carry):
        base = row0 + c * CHUNK
        pltpu.sync_copy(x_hbm.at[pl.ds(base, CHUNK)], x_v)

        copies = []
        for j in range(EMB_DIM):
            copies.append(
                pltpu.async_copy(
                    x_v,
                    out_hbm.at[j, pl.ds(base, CHUNK)],
                    sem,
                )
            )

        @pl.when(c > 0)
        def _():
            pltpu.make_async_copy(
                patch_v,
                out_hbm.at[
                    pl.ds(0, EMB_DIM), pl.ds(base, CHUNK), pl.ds(RMIN, L)
                ],
                psem,
            ).wait()

        def row_body(r, rcarry):
            xv = x_v[r, pl.ds(RMIN, L)]
            ci = xv.astype(jnp.int32) * EMB_DIM
            for j in range(EMB_DIM):
                emb = plsc.load_gather(cb_v, [ci + j])
                patch_v[j, r] = jnp.where(mask_patch, emb, xv)
            return rcarry

        lax.fori_loop(0, CHUNK, row_body, 0)

        for cp in copies:
            cp.wait()
        pltpu.async_copy(
            patch_v,
            out_hbm.at[pl.ds(0, EMB_DIM), pl.ds(base, CHUNK), pl.ds(RMIN, L)],
            psem,
        )
        return carry

    lax.fori_loop(0, NCHUNKS, chunk_body, 0)
    last = row0 + (NCHUNKS - 1) * CHUNK
    pltpu.make_async_copy(
        patch_v,
        out_hbm.at[pl.ds(0, EMB_DIM), pl.ds(last, CHUNK), pl.ds(RMIN, L)],
        psem,
    ).wait()


@jax.jit
def _run(x, cbf):
    fn = functools.partial(
        pl.kernel,
        mesh=plsc.VectorSubcoreMesh(core_axis_name="c", subcore_axis_name="s"),
        compiler_params=pltpu.CompilerParams(
            needs_layout_passes=False, use_tc_tiling_on_sc=False
        ),
        out_type=jax.ShapeDtypeStruct((EMB_DIM, BATCH, IN_DIM), jnp.float32),
        scratch_types=[
            pltpu.VMEM((CHUNK, IN_DIM), jnp.float32),
            pltpu.VMEM((52 * EMB_DIM,), jnp.float32),
            pltpu.VMEM((EMB_DIM, CHUNK, L), jnp.float32),
            pltpu.SemaphoreType.DMA,
            pltpu.SemaphoreType.DMA,
        ],
    )(_sc_planes)
    return fn(x, cbf)


def kernel(x, card_buffer):
    if x.ndim == 3:
        x = x[:, 0, :]
    cbf = card_buffer.reshape(52 * EMB_DIM)
    out = _run(x, cbf)
    return out.transpose(1, 2, 0)
